# baseline (speedup 1.0000x reference)
import functools

import jax
import jax.numpy as jnp
from jax import lax
from jax.experimental import pallas as pl
from jax.experimental.pallas import tpu as pltpu
from jax.experimental.pallas import tpu_sc as plsc

VOCAB = 1000000
D = 64
R = 16384
C = 50
NC, NS = 2, 16
NW = NC * NS
CH = 128
N_CHUNKS = R * C // CH
CH_PER_W = N_CHUNKS // NW
RB = R // CH
NBUF = 4
SCALE = 8.0

_mesh = plsc.VectorSubcoreMesh(
    core_axis_name="c", subcore_axis_name="s", num_cores=NC, num_subcores=NS
)


@functools.partial(
    pl.kernel,
    out_type=jax.ShapeDtypeStruct((C, D // 8, RB, 8, CH), jnp.float32),
    mesh=_mesh,
    scratch_types=[
        pltpu.VMEM((CH_PER_W, CH), jnp.int32),
        pltpu.VMEM((NBUF, CH, D), jnp.float32),
        pltpu.VMEM((2, D, CH), jnp.float32),
        pltpu.SemaphoreType.DMA,
        pltpu.SemaphoreType.DMA,
    ],
    compiler_params=pltpu.CompilerParams(
        use_tc_tiling_on_sc=False, needs_layout_passes=False
    ),
)
def _emb_kernel(idx_hbm, table_hbm, out_hbm, idx_v, bufs, obuf, gsem, ssem):
    wid = lax.axis_index("s") * NC + lax.axis_index("c")
    k0 = wid * CH_PER_W

    pltpu.sync_copy(idx_hbm.at[pl.ds(k0, CH_PER_W)], idx_v)

    for b in range(NBUF - 1):
        pltpu.async_copy(table_hbm.at[idx_v.at[b]], bufs.at[b], gsem)

    row_idx = [jax.lax.iota(jnp.int32, 16) + rg * 16 for rg in range(CH // 16)]

    def do_chunk(j, b):
        buf = bufs.at[b]
        jj = j % 2
        k = k0 + j
        col = k // RB
        rb = k % RB

        @pl.when(j + NBUF - 1 < CH_PER_W)
        def _():
            pltpu.async_copy(
                table_hbm.at[idx_v.at[j + NBUF - 1]],
                bufs.at[(b + NBUF - 1) % NBUF],
                gsem,
            )

        @pl.when(j >= 2)
        def _():
            for db in range(D // 8):
                pltpu.make_async_copy(
                    obuf.at[jj, pl.ds(0, 8)], out_hbm.at[0, db, 0], ssem
                ).wait()

        pltpu.make_async_copy(table_hbm.at[pl.ds(0, CH)], buf, gsem).wait()

        def tr_col(d, carry):
            cvec = jnp.full((16,), d, dtype=jnp.int32)
            for rg in range(CH // 16):
                vals = plsc.load_gather(buf, [row_idx[rg], cvec])
                obuf[jj, d, pl.ds(rg * 16, 16)] = vals * SCALE
            return carry

        lax.fori_loop(0, D, tr_col, 0)

        for db in range(D // 8):
            pltpu.async_copy(
                obuf.at[jj, pl.ds(db * 8, 8)], out_hbm.at[col, db, rb], ssem
            )

    def outer(g, carry):
        for b in range(NBUF):
            do_chunk(g * NBUF + b, b)
        return carry

    lax.fori_loop(0, CH_PER_W // NBUF, outer, 0)

    for _ in range(2):
        for db in range(D // 8):
            pltpu.make_async_copy(
                obuf.at[0, pl.ds(0, 8)], out_hbm.at[0, db, 0], ssem
            ).wait()


def kernel(x, table):
    idx = x.T.astype(jnp.int32).reshape(N_CHUNKS, CH)
    out5 = _emb_kernel(idx, table)
    return out5.transpose(2, 4, 0, 1, 3).reshape(R, C, D)


# speedup vs baseline: 1.6946x; 1.6946x over previous
import functools

import jax
import jax.numpy as jnp
from jax import lax
from jax.experimental import pallas as pl
from jax.experimental.pallas import tpu as pltpu
from jax.experimental.pallas import tpu_sc as plsc

VOCAB = 1000000
D = 64
R = 16384
C = 50
NC, NS = 2, 16
NW = NC * NS
CH = 128
N_CHUNKS = R * C // CH
CH_PER_W = N_CHUNKS // NW
RB = R // CH
NBUF = 4
SCALE = 8.0

_mesh = plsc.VectorSubcoreMesh(
    core_axis_name="c", subcore_axis_name="s", num_cores=NC, num_subcores=NS
)


@functools.partial(
    pl.kernel,
    out_type=jax.ShapeDtypeStruct((C, D // 8, RB, 8, CH), jnp.float32),
    mesh=_mesh,
    scratch_types=[
        pltpu.VMEM((CH_PER_W, CH), jnp.int32),
        pltpu.VMEM((NBUF, CH, D), jnp.float32),
        pltpu.VMEM((2, D // 8, 8, CH), jnp.float32),
        pltpu.SemaphoreType.DMA,
        pltpu.SemaphoreType.DMA,
    ],
    compiler_params=pltpu.CompilerParams(
        use_tc_tiling_on_sc=False, needs_layout_passes=False
    ),
)
def _emb_kernel(idx_hbm, table_hbm, out_hbm, idx_v, bufs, obuf, gsem, ssem):
    wid = lax.axis_index("s") * NC + lax.axis_index("c")
    k0 = wid * CH_PER_W

    pltpu.sync_copy(idx_hbm.at[pl.ds(k0, CH_PER_W)], idx_v)

    for b in range(NBUF - 1):
        pltpu.async_copy(table_hbm.at[idx_v.at[b]], bufs.at[b], gsem)

    iota = lax.iota(jnp.int32, 16)
    rot = [lax.rem(iota + k, 16) for k in range(16)]

    def do_chunk(j, b):
        buf = bufs.at[b]
        jj = j % 2
        jvec = jnp.full((16,), j % 2, dtype=jnp.int32)
        k = k0 + j
        col = k // RB
        rb = k % RB

        @pl.when(j + NBUF - 1 < CH_PER_W)
        def _():
            pltpu.async_copy(
                table_hbm.at[idx_v.at[j + NBUF - 1]],
                bufs.at[(b + NBUF - 1) % NBUF],
                gsem,
            )

        @pl.when(j >= 2)
        def _():
            pltpu.make_async_copy(
                obuf.at[jj], out_hbm.at[0, :, 0], ssem
            ).wait()

        pltpu.make_async_copy(table_hbm.at[pl.ds(0, CH)], buf, gsem).wait()

        def tr_block(blk, carry):
            r0 = (blk % 8) * 16
            d0 = (blk // 8) * 16
            rvec = iota + r0
            for kk in range(16):
                dvec = rot[kk] + d0
                vals = plsc.load_gather(buf, [rvec, dvec])
                plsc.store_scatter(
                    obuf,
                    [jvec, lax.shift_right_logical(dvec, 3),
                     lax.bitwise_and(dvec, 7), rvec],
                    vals * SCALE,
                )
            return carry

        lax.fori_loop(0, (CH // 16) * (D // 16), tr_block, 0)

        pltpu.async_copy(obuf.at[jj], out_hbm.at[col, :, rb], ssem)

    def outer(g, carry):
        for b in range(NBUF):
            do_chunk(g * NBUF + b, b)
        return carry

    lax.fori_loop(0, CH_PER_W // NBUF, outer, 0)

    for _ in range(2):
        pltpu.make_async_copy(obuf.at[0], out_hbm.at[0, :, 0], ssem).wait()


def kernel(x, table):
    idx = x.T.astype(jnp.int32).reshape(N_CHUNKS, CH)
    out5 = _emb_kernel(idx, table)
    return out5.transpose(2, 4, 0, 1, 3).reshape(R, C, D)
